# baseline (device time: 196299 ns/iter reference)
import jax
import jax.numpy as jnp
from jax import lax
from jax.experimental import pallas as pl
from jax.experimental.pallas import tpu as pltpu

N_DEV = 8


def _ring(p):
    return jnp.where(p < 4, p, 11 - p)


def kernel(x, w_mat):
    m, k = x.shape
    _, n = w_mat.shape
    m_out = m // N_DEV

    def body(x_ref, w_ref, out_ref, send_ref, recv_ref, maxsrc_ref,
             maxbuf_ref, send_sems, recv_sems, msend_sems, mrecv_sems):
        d = lax.axis_index("i")
        my_pos = _ring(d)
        right = _ring((my_pos + 1) % N_DEV)
        left = _ring((my_pos - 1) % N_DEV)

        barrier_sem = pltpu.get_barrier_semaphore()
        for nbr in (left, right):
            pl.semaphore_signal(
                barrier_sem, inc=1,
                device_id=(nbr,), device_id_type=pl.DeviceIdType.MESH,
            )
        pl.semaphore_wait(barrier_sem, 2)

        for s in range(N_DEV - 1):
            chunk_log = _ring((my_pos - 1 - s) % N_DEV)
            part = jnp.dot(
                x_ref[pl.ds(chunk_log * m_out, m_out), :], w_ref[...],
                preferred_element_type=jnp.float32,
            )
            if s > 0:
                part = part + recv_ref[s - 1].astype(jnp.float32)
            send_ref[...] = part.astype(jnp.bfloat16)
            rdma = pltpu.make_async_remote_copy(
                src_ref=send_ref,
                dst_ref=recv_ref.at[s],
                send_sem=send_sems.at[s],
                recv_sem=recv_sems.at[s],
                device_id=(right,),
                device_id_type=pl.DeviceIdType.MESH,
            )
            rdma.start()
            rdma.wait()

        part = jnp.dot(
            x_ref[pl.ds(d * m_out, m_out), :], w_ref[...],
            preferred_element_type=jnp.float32,
        )
        y = jnp.maximum(part + recv_ref[N_DEV - 2].astype(jnp.float32), 0.0)
        lmax = jnp.max(y)
        maxsrc_ref[...] = jnp.full((8, 128), lmax, jnp.float32)

        rdmas = []
        for t in range(1, N_DEV):
            dst = _ring((my_pos + t) % N_DEV)
            r = pltpu.make_async_remote_copy(
                src_ref=maxsrc_ref,
                dst_ref=maxbuf_ref.at[t - 1],
                send_sem=msend_sems.at[t - 1],
                recv_sem=mrecv_sems.at[t - 1],
                device_id=(dst,),
                device_id_type=pl.DeviceIdType.MESH,
            )
            r.start()
            rdmas.append(r)
        for r in rdmas:
            r.wait_send()
        for r in rdmas:
            r.wait_recv()
        gmax = jnp.maximum(jnp.max(maxbuf_ref[...]), lmax)

        scale = gmax / 127.0
        q = jnp.minimum(jnp.round(y / scale), 127.0)
        out_ref[...] = q * scale

    return pl.pallas_call(
        body,
        out_shape=jax.ShapeDtypeStruct((m_out, n), jnp.float32),
        in_specs=[
            pl.BlockSpec(memory_space=pltpu.VMEM),
            pl.BlockSpec(memory_space=pltpu.VMEM),
        ],
        out_specs=pl.BlockSpec(memory_space=pltpu.VMEM),
        scratch_shapes=[
            pltpu.VMEM((m_out, n), jnp.bfloat16),
            pltpu.VMEM((N_DEV - 1, m_out, n), jnp.bfloat16),
            pltpu.VMEM((8, 128), jnp.float32),
            pltpu.VMEM((N_DEV - 1, 8, 128), jnp.float32),
            pltpu.SemaphoreType.DMA((N_DEV - 1,)),
            pltpu.SemaphoreType.DMA((N_DEV - 1,)),
            pltpu.SemaphoreType.DMA((N_DEV - 1,)),
            pltpu.SemaphoreType.DMA((N_DEV - 1,)),
        ],
        compiler_params=pltpu.CompilerParams(collective_id=0),
    )(x, w_mat)


# device time: 119296 ns/iter; 1.6455x vs baseline; 1.6455x over previous
import jax
import jax.numpy as jnp
from jax import lax
from jax.experimental import pallas as pl
from jax.experimental.pallas import tpu as pltpu

N_DEV = 8


def _ring(p):
    return jnp.where(p < 4, p, 11 - p)


def kernel(x, w_mat):
    m, k = x.shape
    _, n = w_mat.shape
    m_out = m // N_DEV

    n_half = n // 2

    def body(x_ref, w_ref, out_ref, scw_ref, sccw_ref, rcw_ref, rccw_ref,
             maxsrc_ref, maxbuf_ref, cw_send_sems, cw_recv_sems,
             ccw_send_sems, ccw_recv_sems, msend_sems, mrecv_sems):
        d = lax.axis_index("i")
        my_pos = _ring(d)
        right = _ring((my_pos + 1) % N_DEV)
        left = _ring((my_pos - 1) % N_DEV)

        barrier_sem = pltpu.get_barrier_semaphore()
        for nbr in (left, right):
            pl.semaphore_signal(
                barrier_sem, inc=1,
                device_id=(nbr,), device_id_type=pl.DeviceIdType.MESH,
            )
        pl.semaphore_wait(barrier_sem, 2)

        def cw_rows(s):
            return _ring((my_pos - 1 - s) % N_DEV) * m_out

        def ccw_rows(s):
            return _ring((my_pos + 1 + s) % N_DEV) * m_out

        scw_ref[0] = jnp.dot(
            x_ref[pl.ds(cw_rows(0), m_out), :], w_ref[:, :n_half],
            preferred_element_type=jnp.float32,
        ).astype(jnp.bfloat16)
        sccw_ref[0] = jnp.dot(
            x_ref[pl.ds(ccw_rows(0), m_out), :], w_ref[:, n_half:],
            preferred_element_type=jnp.float32,
        ).astype(jnp.bfloat16)

        descs = []
        y = None
        for s in range(N_DEV - 1):
            slot = s % 2
            cw = pltpu.make_async_remote_copy(
                src_ref=scw_ref.at[slot],
                dst_ref=rcw_ref.at[s],
                send_sem=cw_send_sems.at[s],
                recv_sem=cw_recv_sems.at[s],
                device_id=(right,),
                device_id_type=pl.DeviceIdType.MESH,
            )
            ccw = pltpu.make_async_remote_copy(
                src_ref=sccw_ref.at[slot],
                dst_ref=rccw_ref.at[s],
                send_sem=ccw_send_sems.at[s],
                recv_sem=ccw_recv_sems.at[s],
                device_id=(left,),
                device_id_type=pl.DeviceIdType.MESH,
            )
            cw.start()
            ccw.start()
            descs.append((cw, ccw))

            if s < N_DEV - 2:
                row_a, row_b = cw_rows(s + 1), ccw_rows(s + 1)
            else:
                row_a = row_b = d * m_out
            pa = jnp.dot(
                x_ref[pl.ds(row_a, m_out), :], w_ref[:, :n_half],
                preferred_element_type=jnp.float32,
            )
            pb = jnp.dot(
                x_ref[pl.ds(row_b, m_out), :], w_ref[:, n_half:],
                preferred_element_type=jnp.float32,
            )

            cw.wait_recv()
            ccw.wait_recv()
            acc_a = pa + rcw_ref[s].astype(jnp.float32)
            acc_b = pb + rccw_ref[s].astype(jnp.float32)
            if s < N_DEV - 2:
                if s >= 1:
                    descs[s - 1][0].wait_send()
                    descs[s - 1][1].wait_send()
                scw_ref[(s + 1) % 2] = acc_a.astype(jnp.bfloat16)
                sccw_ref[(s + 1) % 2] = acc_b.astype(jnp.bfloat16)
            else:
                y = jnp.maximum(
                    jnp.concatenate([acc_a, acc_b], axis=1), 0.0
                )
        for i in (N_DEV - 3, N_DEV - 2):
            descs[i][0].wait_send()
            descs[i][1].wait_send()

        lmax = jnp.max(y)
        maxsrc_ref[...] = jnp.full((8, 128), lmax, jnp.float32)

        rdmas = []
        for t in range(1, N_DEV):
            dst = _ring((my_pos + t) % N_DEV)
            r = pltpu.make_async_remote_copy(
                src_ref=maxsrc_ref,
                dst_ref=maxbuf_ref.at[t - 1],
                send_sem=msend_sems.at[t - 1],
                recv_sem=mrecv_sems.at[t - 1],
                device_id=(dst,),
                device_id_type=pl.DeviceIdType.MESH,
            )
            r.start()
            rdmas.append(r)
        for r in rdmas:
            r.wait_send()
        for r in rdmas:
            r.wait_recv()
        gmax = jnp.maximum(jnp.max(maxbuf_ref[...]), lmax)

        scale = gmax / 127.0
        q = jnp.minimum(jnp.round(y / scale), 127.0)
        out_ref[...] = q * scale

    return pl.pallas_call(
        body,
        out_shape=jax.ShapeDtypeStruct((m_out, n), jnp.float32),
        in_specs=[
            pl.BlockSpec(memory_space=pltpu.VMEM),
            pl.BlockSpec(memory_space=pltpu.VMEM),
        ],
        out_specs=pl.BlockSpec(memory_space=pltpu.VMEM),
        scratch_shapes=[
            pltpu.VMEM((2, m_out, n // 2), jnp.bfloat16),
            pltpu.VMEM((2, m_out, n // 2), jnp.bfloat16),
            pltpu.VMEM((N_DEV - 1, m_out, n // 2), jnp.bfloat16),
            pltpu.VMEM((N_DEV - 1, m_out, n // 2), jnp.bfloat16),
            pltpu.VMEM((8, 128), jnp.float32),
            pltpu.VMEM((N_DEV - 1, 8, 128), jnp.float32),
            pltpu.SemaphoreType.DMA((N_DEV - 1,)),
            pltpu.SemaphoreType.DMA((N_DEV - 1,)),
            pltpu.SemaphoreType.DMA((N_DEV - 1,)),
            pltpu.SemaphoreType.DMA((N_DEV - 1,)),
            pltpu.SemaphoreType.DMA((N_DEV - 1,)),
            pltpu.SemaphoreType.DMA((N_DEV - 1,)),
        ],
        compiler_params=pltpu.CompilerParams(collective_id=0),
    )(x, w_mat)


# device time: 99441 ns/iter; 1.9740x vs baseline; 1.1997x over previous
import jax
import jax.numpy as jnp
from jax import lax
from jax.experimental import pallas as pl
from jax.experimental.pallas import tpu as pltpu

N_DEV = 8
N_SEG = 4


def _ring(p):
    return jnp.where(p < 4, p, 11 - p)


def kernel(x, w_mat):
    m, k = x.shape
    _, n = w_mat.shape
    m_out = m // N_DEV
    n_seg = n // N_SEG

    def body(x_ref, w_ref, out_ref, maxsrc_ref, maxbuf_ref,
             s0_ref, s1_ref, s2_ref, s3_ref, r0_ref, r1_ref, r2_ref, r3_ref,
             ss0, ss1, ss2, ss3, rs0, rs1, rs2, rs3, msend_sems, mrecv_sems):
        d = lax.axis_index("i")
        my_pos = _ring(d)
        right = _ring((my_pos + 1) % N_DEV)
        left = _ring((my_pos - 1) % N_DEV)

        barrier_sem = pltpu.get_barrier_semaphore()
        for nbr in (left, right):
            pl.semaphore_signal(
                barrier_sem, inc=1,
                device_id=(nbr,), device_id_type=pl.DeviceIdType.MESH,
            )
        pl.semaphore_wait(barrier_sem, 2)

        def cw_chunk(s):
            return jnp.where(s < N_DEV - 1, _ring((my_pos - 1 - s) % N_DEV), d)

        def ccw_chunk(s):
            return jnp.where(s < N_DEV - 1, _ring((my_pos + 1 + s) % N_DEV), d)

        segs = [
            (s0_ref, r0_ref, ss0, rs0, 0 * n_seg, right, cw_chunk),
            (s2_ref, r2_ref, ss2, rs2, 2 * n_seg, left, ccw_chunk),
            (s1_ref, r1_ref, ss1, rs1, 1 * n_seg, right, cw_chunk),
            (s3_ref, r3_ref, ss3, rs3, 3 * n_seg, left, ccw_chunk),
        ]

        def seg_dot(seg_i, s):
            _, _, _, _, col, _, chunk = segs[seg_i]
            return jnp.dot(
                x_ref[pl.ds(chunk(s) * m_out, m_out), :],
                w_ref[:, col:col + n_seg],
                preferred_element_type=jnp.float32,
            )

        part = [seg_dot(i, 0) for i in range(N_SEG)]
        descs = [[] for _ in range(N_SEG)]
        for s in range(N_DEV - 1):
            slot = s % 2
            for i, (snd, rcv, ssem, rsem, col, dst, chunk) in enumerate(segs):
                if s == 0:
                    snd[slot] = part[i].astype(jnp.bfloat16)
                else:
                    descs[i][s - 1].wait_recv()
                    if s >= 2:
                        descs[i][s - 2].wait_send()
                    snd[slot] = (
                        part[i] + rcv[s - 1].astype(jnp.float32)
                    ).astype(jnp.bfloat16)
                rdma = pltpu.make_async_remote_copy(
                    src_ref=snd.at[slot],
                    dst_ref=rcv.at[s],
                    send_sem=ssem.at[s],
                    recv_sem=rsem.at[s],
                    device_id=(dst,),
                    device_id_type=pl.DeviceIdType.MESH,
                )
                rdma.start()
                descs[i].append(rdma)
                part[i] = seg_dot(i, s + 1)

        finals = [None] * N_SEG
        for i, (snd, rcv, ssem, rsem, col, dst, chunk) in enumerate(segs):
            descs[i][N_DEV - 2].wait_recv()
            finals[i] = part[i] + rcv[N_DEV - 2].astype(jnp.float32)
            descs[i][N_DEV - 3].wait_send()
            descs[i][N_DEV - 2].wait_send()
        y = jnp.maximum(
            jnp.concatenate([finals[0], finals[2], finals[1], finals[3]],
                            axis=1),
            0.0,
        )
        lmax = jnp.max(y)
        maxsrc_ref[...] = jnp.full((8, 128), lmax, jnp.float32)

        rdmas = []
        for t in range(1, N_DEV):
            dst = _ring((my_pos + t) % N_DEV)
            r = pltpu.make_async_remote_copy(
                src_ref=maxsrc_ref,
                dst_ref=maxbuf_ref.at[t - 1],
                send_sem=msend_sems.at[t - 1],
                recv_sem=mrecv_sems.at[t - 1],
                device_id=(dst,),
                device_id_type=pl.DeviceIdType.MESH,
            )
            r.start()
            rdmas.append(r)
        for r in rdmas:
            r.wait_send()
        for r in rdmas:
            r.wait_recv()
        gmax = jnp.maximum(jnp.max(maxbuf_ref[...]), lmax)

        scale = gmax / 127.0
        q = jnp.minimum(jnp.round(y / scale), 127.0)
        out_ref[...] = q * scale

    seg_send = pltpu.VMEM((2, m_out, n_seg), jnp.bfloat16)
    seg_recv = pltpu.VMEM((N_DEV - 1, m_out, n_seg), jnp.bfloat16)
    seg_sems = pltpu.SemaphoreType.DMA((N_DEV - 1,))
    return pl.pallas_call(
        body,
        out_shape=jax.ShapeDtypeStruct((m_out, n), jnp.float32),
        in_specs=[
            pl.BlockSpec(memory_space=pltpu.VMEM),
            pl.BlockSpec(memory_space=pltpu.VMEM),
        ],
        out_specs=pl.BlockSpec(memory_space=pltpu.VMEM),
        scratch_shapes=[
            pltpu.VMEM((8, 128), jnp.float32),
            pltpu.VMEM((N_DEV - 1, 8, 128), jnp.float32),
            seg_send, seg_send, seg_send, seg_send,
            seg_recv, seg_recv, seg_recv, seg_recv,
            seg_sems, seg_sems, seg_sems, seg_sems,
            seg_sems, seg_sems, seg_sems, seg_sems,
            pltpu.SemaphoreType.DMA((N_DEV - 1,)),
            pltpu.SemaphoreType.DMA((N_DEV - 1,)),
        ],
        compiler_params=pltpu.CompilerParams(collective_id=0),
    )(x, w_mat)


# device time: 82384 ns/iter; 2.3827x vs baseline; 1.2070x over previous
import jax
import jax.numpy as jnp
from jax import lax
from jax.experimental import pallas as pl
from jax.experimental.pallas import tpu as pltpu

N_DEV = 8
M_OUT = 512
G_OFF = (0, 768, 1408)
G_COLS = (768, 640, 640)
G_DIMS = ((0, 1, 2), (1, 2, 0), (2, 0, 1))
OTHER = {0: (1, 2), 1: (0, 2), 2: (0, 1)}


def _ring(p):
    return jnp.where(p < 4, p, 11 - p)


def kernel(x, w_mat):
    m, k = x.shape
    _, n = w_mat.shape

    def body(x_ref, w_ref, out_ref, maxsrc_ref, maxbuf_ref,
             p0, p1, p2, rb1_0, rb1_1, rb1_2, rb2_0, rb2_1, rb2_2,
             rb3_0, rb3_1, rb3_2, ss0, ss1, ss2, rs0, rs1, rs2,
             msend_sems, mrecv_sems):
        d = lax.axis_index("i")
        m4 = d % 4
        mybits = [
            jnp.where((m4 == 1) | (m4 == 2), 1, 0),
            jnp.where(m4 >= 2, 1, 0),
            jnp.where(d >= 4, 1, 0),
        ]

        def cid(bits):
            return jnp.where(bits[1] == 0, bits[0], 3 - bits[0]) + 4 * bits[2]

        def flip(bits, dim):
            b = list(bits)
            b[dim] = 1 - b[dim]
            return b

        neighbors = [cid(flip(mybits, dim)) for dim in range(3)]

        barrier_sem = pltpu.get_barrier_semaphore()
        for nbr in neighbors:
            pl.semaphore_signal(
                barrier_sem, inc=1,
                device_id=(nbr,), device_id_type=pl.DeviceIdType.MESH,
            )
        pl.semaphore_wait(barrier_sem, 3)

        P = [p0, p1, p2]
        RB1 = [rb1_0, rb1_1, rb1_2]
        RB2 = [rb2_0, rb2_1, rb2_2]
        RB3 = [rb3_0, rb3_1, rb3_2]
        SS = [ss0, ss1, ss2]
        RS = [rs0, rs1, rs2]

        def r1_chunk(g, j, side_bit):
            d1, d2, d3 = G_DIMS[g]
            bits = [None, None, None]
            bits[d1] = side_bit
            bits[d2] = mybits[d2] ^ (1 if j < 2 else 0)
            bits[d3] = mybits[d3] ^ (j & 1)
            return cid(bits)

        def r2_chunk(g, j, side_bit):
            d1, d2, d3 = G_DIMS[g]
            bits = [None, None, None]
            bits[d1] = mybits[d1]
            bits[d2] = side_bit
            bits[d3] = mybits[d3] ^ (1 if j == 0 else 0)
            return cid(bits)

        def dot_chunk(g, ck):
            return jnp.dot(
                x_ref[pl.ds(ck * M_OUT, M_OUT), :],
                w_ref[:, G_OFF[g]:G_OFF[g] + G_COLS[g]],
                preferred_element_type=jnp.float32,
            )

        def rdma(g, sem_idx, src_row, dst_ref, dim):
            return pltpu.make_async_remote_copy(
                src_ref=P[g].at[pl.ds(src_row * M_OUT, M_OUT)],
                dst_ref=dst_ref,
                send_sem=SS[g].at[sem_idx],
                recv_sem=RS[g].at[sem_idx],
                device_id=(neighbors[dim],),
                device_id_type=pl.DeviceIdType.MESH,
            )

        send_descs = []

        r1_descs = [[None] * 4 for _ in range(3)]
        for j in range(4):
            for g in range(3):
                dim = G_DIMS[g][0]
                ck = r1_chunk(g, j, 1 - mybits[dim])
                P[g][pl.ds(ck * M_OUT, M_OUT)] = (
                    dot_chunk(g, ck).astype(jnp.bfloat16)
                )
                r = rdma(g, j, ck, RB1[g].at[j], dim)
                r.start()
                r1_descs[g][j] = r
                send_descs.append(r)

        for j in range(4):
            for g in range(3):
                dim = G_DIMS[g][0]
                ck = r1_chunk(g, j, mybits[dim])
                P[g][pl.ds(ck * M_OUT, M_OUT)] = (
                    dot_chunk(g, ck).astype(jnp.bfloat16)
                )

        order = (1, 2, 0)

        r2_descs = [[None] * 2 for _ in range(3)]
        for g in order:
            d1, d2, d3 = G_DIMS[g]
            for j in range(4):
                r1_descs[g][j].wait_recv()
                ck = r1_chunk(g, j, mybits[d1])
                row = pl.ds(ck * M_OUT, M_OUT)
                P[g][row] = (
                    P[g][row].astype(jnp.float32)
                    + RB1[g][j].astype(jnp.float32)
                ).astype(jnp.bfloat16)
                if j == 1:
                    for sj in range(2):
                        sck = r2_chunk(g, sj, 1 - mybits[d2])
                        r = rdma(g, 4 + sj, sck, RB2[g].at[sj], d2)
                        r.start()
                        r2_descs[g][sj] = r
                        send_descs.append(r)

        r3_descs = [None] * 3
        for g in order:
            d1, d2, d3 = G_DIMS[g]
            for sj in range(2):
                r2_descs[g][sj].wait_recv()
                ck = r2_chunk(g, sj, mybits[d2])
                row = pl.ds(ck * M_OUT, M_OUT)
                P[g][row] = (
                    P[g][row].astype(jnp.float32)
                    + RB2[g][sj].astype(jnp.float32)
                ).astype(jnp.bfloat16)
            sck = cid(flip(mybits, d3))
            r = rdma(g, 6, sck, RB3[g].at[0], d3)
            r.start()
            r3_descs[g] = r
            send_descs.append(r)

        finals = [None] * 3
        for g in order:
            r3_descs[g].wait_recv()
            finals[g] = (
                P[g][pl.ds(d * M_OUT, M_OUT)].astype(jnp.float32)
                + RB3[g][0].astype(jnp.float32)
            )
        y = jnp.maximum(
            jnp.concatenate([finals[0], finals[1], finals[2]], axis=1), 0.0
        )
        lmax = jnp.max(y)
        maxsrc_ref[...] = jnp.full((8, 128), lmax, jnp.float32)

        my_pos = _ring(d)
        rdmas = []
        for t in range(1, N_DEV):
            dst = _ring((my_pos + t) % N_DEV)
            r = pltpu.make_async_remote_copy(
                src_ref=maxsrc_ref,
                dst_ref=maxbuf_ref.at[t - 1],
                send_sem=msend_sems.at[t - 1],
                recv_sem=mrecv_sems.at[t - 1],
                device_id=(dst,),
                device_id_type=pl.DeviceIdType.MESH,
            )
            r.start()
            rdmas.append(r)
        for r in send_descs:
            r.wait_send()
        for r in rdmas:
            r.wait_send()
        for r in rdmas:
            r.wait_recv()
        gmax = jnp.maximum(jnp.max(maxbuf_ref[...]), lmax)

        scale = gmax / 127.0
        q = jnp.minimum(jnp.round(y / scale), 127.0)
        out_ref[...] = q * scale

    def _dyn_slot(ref, j):
        return ref[j]

    scratch = [
        pltpu.VMEM((8, 128), jnp.float32),
        pltpu.VMEM((N_DEV - 1, 8, 128), jnp.float32),
    ]
    for g in range(3):
        scratch.append(pltpu.VMEM((m, G_COLS[g]), jnp.bfloat16))
    for g in range(3):
        scratch.append(pltpu.VMEM((4, M_OUT, G_COLS[g]), jnp.bfloat16))
    for g in range(3):
        scratch.append(pltpu.VMEM((2, M_OUT, G_COLS[g]), jnp.bfloat16))
    for g in range(3):
        scratch.append(pltpu.VMEM((1, M_OUT, G_COLS[g]), jnp.bfloat16))
    for _ in range(3):
        scratch.append(pltpu.SemaphoreType.DMA((7,)))
    for _ in range(3):
        scratch.append(pltpu.SemaphoreType.DMA((7,)))
    scratch.append(pltpu.SemaphoreType.DMA((N_DEV - 1,)))
    scratch.append(pltpu.SemaphoreType.DMA((N_DEV - 1,)))

    return pl.pallas_call(
        body,
        out_shape=jax.ShapeDtypeStruct((M_OUT, n), jnp.float32),
        in_specs=[
            pl.BlockSpec(memory_space=pltpu.VMEM),
            pl.BlockSpec(memory_space=pltpu.VMEM),
        ],
        out_specs=pl.BlockSpec(memory_space=pltpu.VMEM),
        scratch_shapes=scratch,
        compiler_params=pltpu.CompilerParams(
            collective_id=0, vmem_limit_bytes=64 * 1024 * 1024
        ),
    )(x, w_mat)


# device time: 81496 ns/iter; 2.4087x vs baseline; 1.0109x over previous
import jax
import jax.numpy as jnp
from jax import lax
from jax.experimental import pallas as pl
from jax.experimental.pallas import tpu as pltpu

N_DEV = 8
M_OUT = 512
G_OFF = (0, 768, 1408)
G_COLS = (768, 640, 640)
G_DIMS = ((0, 1, 2), (1, 2, 0), (2, 0, 1))
OTHER = {0: (1, 2), 1: (0, 2), 2: (0, 1)}


def _ring(p):
    return jnp.where(p < 4, p, 11 - p)


def kernel(x, w_mat):
    m, k = x.shape
    _, n = w_mat.shape

    def body(x_ref, w_ref, out_ref, maxsrc_ref, maxbuf_ref,
             p0, p1, p2, rb1_0, rb1_1, rb1_2, rb2_0, rb2_1, rb2_2,
             rb3_0, rb3_1, rb3_2, ss0, ss1, ss2, rs0, rs1, rs2,
             msend_sems, mrecv_sems):
        d = lax.axis_index("i")
        m4 = d % 4
        mybits = [
            jnp.where((m4 == 1) | (m4 == 2), 1, 0),
            jnp.where(m4 >= 2, 1, 0),
            jnp.where(d >= 4, 1, 0),
        ]

        def cid(bits):
            return jnp.where(bits[1] == 0, bits[0], 3 - bits[0]) + 4 * bits[2]

        def flip(bits, dim):
            b = list(bits)
            b[dim] = 1 - b[dim]
            return b

        neighbors = [cid(flip(mybits, dim)) for dim in range(3)]

        barrier_sem = pltpu.get_barrier_semaphore()
        for nbr in neighbors:
            pl.semaphore_signal(
                barrier_sem, inc=1,
                device_id=(nbr,), device_id_type=pl.DeviceIdType.MESH,
            )
        pl.semaphore_wait(barrier_sem, 3)

        P = [p0, p1, p2]
        RB1 = [rb1_0, rb1_1, rb1_2]
        RB2 = [rb2_0, rb2_1, rb2_2]
        RB3 = [rb3_0, rb3_1, rb3_2]
        SS = [ss0, ss1, ss2]
        RS = [rs0, rs1, rs2]

        def r1_chunk(g, j, side_bit):
            d1, d2, d3 = G_DIMS[g]
            bits = [None, None, None]
            bits[d1] = side_bit
            bits[d2] = mybits[d2] ^ (1 if j < 2 else 0)
            bits[d3] = mybits[d3] ^ (j & 1)
            return cid(bits)

        def r2_chunk(g, j, side_bit):
            d1, d2, d3 = G_DIMS[g]
            bits = [None, None, None]
            bits[d1] = mybits[d1]
            bits[d2] = side_bit
            bits[d3] = mybits[d3] ^ (1 if j == 0 else 0)
            return cid(bits)

        def dot_chunk(g, ck):
            return jnp.dot(
                x_ref[pl.ds(ck * M_OUT, M_OUT), :],
                w_ref[:, G_OFF[g]:G_OFF[g] + G_COLS[g]],
                preferred_element_type=jnp.float32,
            )

        def rdma(g, sem_idx, src_row, dst_ref, dim):
            return pltpu.make_async_remote_copy(
                src_ref=P[g].at[pl.ds(src_row * M_OUT, M_OUT)],
                dst_ref=dst_ref,
                send_sem=SS[g].at[sem_idx],
                recv_sem=RS[g].at[sem_idx],
                device_id=(neighbors[dim],),
                device_id_type=pl.DeviceIdType.MESH,
            )

        send_descs = []

        r1_descs = [[None] * 4 for _ in range(3)]
        for j in range(4):
            for g in range(3):
                dim = G_DIMS[g][0]
                ck = r1_chunk(g, j, 1 - mybits[dim])
                P[g][pl.ds(ck * M_OUT, M_OUT)] = (
                    dot_chunk(g, ck).astype(jnp.bfloat16)
                )
                r = rdma(g, j, ck, RB1[g].at[j], dim)
                r.start()
                r1_descs[g][j] = r
                send_descs.append(r)

        for j in range(4):
            for g in range(3):
                dim = G_DIMS[g][0]
                ck = r1_chunk(g, j, mybits[dim])
                P[g][pl.ds(ck * M_OUT, M_OUT)] = (
                    dot_chunk(g, ck).astype(jnp.bfloat16)
                )

        order = (1, 2, 0)

        r2_descs = [[None] * 2 for _ in range(3)]
        for g in order:
            d1, d2, d3 = G_DIMS[g]
            for j in range(4):
                r1_descs[g][j].wait_recv()
                ck = r1_chunk(g, j, mybits[d1])
                row = pl.ds(ck * M_OUT, M_OUT)
                P[g][row] = (
                    P[g][row].astype(jnp.float32)
                    + RB1[g][j].astype(jnp.float32)
                ).astype(jnp.bfloat16)
                if j == 1:
                    for sj in range(2):
                        sck = r2_chunk(g, sj, 1 - mybits[d2])
                        r = rdma(g, 4 + sj, sck, RB2[g].at[sj], d2)
                        r.start()
                        r2_descs[g][sj] = r
                        send_descs.append(r)

        r3_descs = [None] * 3
        for g in order:
            d1, d2, d3 = G_DIMS[g]
            for sj in range(2):
                r2_descs[g][sj].wait_recv()
                ck = r2_chunk(g, sj, mybits[d2])
                row = pl.ds(ck * M_OUT, M_OUT)
                P[g][row] = (
                    P[g][row].astype(jnp.float32)
                    + RB2[g][sj].astype(jnp.float32)
                ).astype(jnp.bfloat16)
            sck = cid(flip(mybits, d3))
            r = rdma(g, 6, sck, RB3[g].at[0], d3)
            r.start()
            r3_descs[g] = r
            send_descs.append(r)

        lmax = jnp.float32(0.0)
        for g in order:
            r3_descs[g].wait_recv()
            yg = jnp.maximum(
                P[g][pl.ds(d * M_OUT, M_OUT)].astype(jnp.float32)
                + RB3[g][0].astype(jnp.float32),
                0.0,
            )
            out_ref[:, G_OFF[g]:G_OFF[g] + G_COLS[g]] = yg
            lmax = jnp.maximum(lmax, jnp.max(yg))
        maxsrc_ref[...] = jnp.full((8, 128), lmax, jnp.float32)

        my_pos = _ring(d)
        rdmas = []
        for t in range(1, N_DEV):
            dst = _ring((my_pos + t) % N_DEV)
            r = pltpu.make_async_remote_copy(
                src_ref=maxsrc_ref,
                dst_ref=maxbuf_ref.at[t - 1],
                send_sem=msend_sems.at[t - 1],
                recv_sem=mrecv_sems.at[t - 1],
                device_id=(dst,),
                device_id_type=pl.DeviceIdType.MESH,
            )
            r.start()
            rdmas.append(r)
        for r in send_descs:
            r.wait_send()
        for r in rdmas:
            r.wait_send()
        for r in rdmas:
            r.wait_recv()
        gmax = jnp.maximum(jnp.max(maxbuf_ref[...]), lmax)

        scale = gmax / 127.0
        q = jnp.minimum(jnp.round(out_ref[...] / scale), 127.0)
        out_ref[...] = q * scale

    def _dyn_slot(ref, j):
        return ref[j]

    scratch = [
        pltpu.VMEM((8, 128), jnp.float32),
        pltpu.VMEM((N_DEV - 1, 8, 128), jnp.float32),
    ]
    for g in range(3):
        scratch.append(pltpu.VMEM((m, G_COLS[g]), jnp.bfloat16))
    for g in range(3):
        scratch.append(pltpu.VMEM((4, M_OUT, G_COLS[g]), jnp.bfloat16))
    for g in range(3):
        scratch.append(pltpu.VMEM((2, M_OUT, G_COLS[g]), jnp.bfloat16))
    for g in range(3):
        scratch.append(pltpu.VMEM((1, M_OUT, G_COLS[g]), jnp.bfloat16))
    for _ in range(3):
        scratch.append(pltpu.SemaphoreType.DMA((7,)))
    for _ in range(3):
        scratch.append(pltpu.SemaphoreType.DMA((7,)))
    scratch.append(pltpu.SemaphoreType.DMA((N_DEV - 1,)))
    scratch.append(pltpu.SemaphoreType.DMA((N_DEV - 1,)))

    return pl.pallas_call(
        body,
        out_shape=jax.ShapeDtypeStruct((M_OUT, n), jnp.float32),
        in_specs=[
            pl.BlockSpec(memory_space=pltpu.VMEM),
            pl.BlockSpec(memory_space=pltpu.VMEM),
        ],
        out_specs=pl.BlockSpec(memory_space=pltpu.VMEM),
        scratch_shapes=scratch,
        compiler_params=pltpu.CompilerParams(
            collective_id=0, vmem_limit_bytes=64 * 1024 * 1024
        ),
    )(x, w_mat)
